# baseline (speedup 1.0000x reference)
import functools

import jax
import jax.numpy as jnp
import numpy as np
from jax import lax
from jax.experimental import pallas as pl
from jax.experimental.pallas import tpu as pltpu
from jax.experimental.pallas import tpu_sc as plsc

_D = 32
_W = 64
_OW = 63
_NB = 16
_NR = 16
_ITEMS = _NB * _NR
_ITEMS_PER_WORKER = _ITEMS // 32
_PLANE = 4 * 4 * 8 * 128
_SCALE = np.float32(np.sqrt(32.0))


def _pos_encoding_32() -> np.ndarray:
    depth = _D / 2
    positions = np.arange(_D)[:, np.newaxis]
    depths = np.arange(depth)[np.newaxis, :] / depth
    angle_rates = 1 / 10000**depths
    angle_rads = positions * angle_rates
    return np.concatenate(
        [np.sin(angle_rads), np.cos(angle_rads)], axis=-1
    ).astype(np.float32)


_PE_BC = np.ascontiguousarray(
    np.broadcast_to(_pos_encoding_32()[:, :, None], (_D, _D, 16))
)


def _sc_body(x_hbm, pe_hbm, table_hbm, out_hbm, xv, gv, ov, pev, tokT, gsem, osem):
    wid = lax.axis_index("s") * 2 + lax.axis_index("c")
    pltpu.sync_copy(pe_hbm, pev)

    iota = lax.iota(jnp.int32, 16)
    for q in range(2):
        for m in range(32):
            pv = jnp.minimum(m * 16 + iota, _OW * 8 - 1)
            i = lax.div(pv, _OW)
            tok = i * _W + (pv - i * _OW) + q
            tokT[q, pl.ds(m * 16, 16)] = tok

    def stage(t, slot):
        item = wid * _ITEMS_PER_WORKER + t
        b = item // _NR
        r = item % _NR
        pltpu.sync_copy(x_hbm.at[b, :, r], xv.at[slot])
        descs = [
            pltpu.make_async_copy(
                table_hbm.at[xv.at[slot, i]],
                gv.at[slot, pl.ds(i * _W, _W)],
                gsem,
            )
            for i in range(8)
        ]
        for d in descs:
            d.start()
        return descs

    def expand(t, slot):
        item = wid * _ITEMS_PER_WORKER + t
        r = item % _NR
        gvs = gv.at[slot]

        @plsc.parallel_loop(0, 2048, unroll=4)
        def body(v):
            q = v >> 10
            d = (v >> 5) & 31
            m = v & 31
            pe = pev[2 * r + q, d, 0:16]
            tokv = tokT[q, pl.ds(m * 16, 16)]
            dsplat = jnp.broadcast_to(d, (16,))
            g = plsc.load_gather(gvs, [tokv, dsplat])
            start = (
                (q << 14)
                + ((d >> 3) << 12)
                + ((d & 7) << 7)
                + ((m >> 3) << 10)
                + ((m & 7) << 4)
            )
            ov[slot, pl.ds(start, 16)] = g * _SCALE + pe

    out_descs = [None, None]
    gdescs = stage(0, 0)
    for t in range(_ITEMS_PER_WORKER):
        slot = t % 2
        for d in gdescs:
            d.wait()
        if t + 1 < _ITEMS_PER_WORKER:
            gdescs = stage(t + 1, 1 - slot)
        if out_descs[slot] is not None:
            out_descs[slot].wait()
        expand(t, slot)
        item = wid * _ITEMS_PER_WORKER + t
        out_descs[slot] = pltpu.make_async_copy(
            ov.at[slot], out_hbm.at[pl.ds(item * 2 * _PLANE, 2 * _PLANE)], osem
        )
        out_descs[slot].start()
    for d in out_descs:
        if d is not None:
            d.wait()


@functools.partial(
    pl.kernel,
    out_type=jax.ShapeDtypeStruct((_ITEMS * 2 * _PLANE,), jnp.float32),
    mesh=plsc.VectorSubcoreMesh(core_axis_name="c", subcore_axis_name="s"),
    scratch_types=[
        pltpu.VMEM((2, 8, _W), jnp.int32),
        pltpu.VMEM((2, 8 * _W, _D), jnp.float32),
        pltpu.VMEM((2, 2 * _PLANE), jnp.float32),
        pltpu.VMEM((_D, _D, 16), jnp.float32),
        pltpu.VMEM((2, 512), jnp.int32),
        pltpu.SemaphoreType.DMA,
        pltpu.SemaphoreType.DMA,
    ],
    compiler_params=pltpu.CompilerParams(
        use_tc_tiling_on_sc=False, needs_layout_passes=False
    ),
)
def _pos_emb_sc(x_hbm, pe_hbm, table_hbm, out_hbm, xv, gv, ov, pev, tokT, gsem, osem):
    _sc_body(x_hbm, pe_hbm, table_hbm, out_hbm, xv, gv, ov, pev, tokT, gsem, osem)


def kernel(x, table):
    x4 = x.reshape(_NB, 8, 16, _W)
    out1d = _pos_emb_sc(x4, _PE_BC, table)
    out6 = out1d.reshape(_NB, _D, 4, 4, 8, 128)
    t = jnp.transpose(out6, (0, 3, 5, 1, 2, 4))
    return t.reshape(_NB, 512, _D, _D)[:, : 8 * _OW]


# speedup vs baseline: 1.7153x; 1.7153x over previous
import functools

import jax
import jax.numpy as jnp
import numpy as np
from jax import lax
from jax.experimental import pallas as pl
from jax.experimental.pallas import tpu as pltpu
from jax.experimental.pallas import tpu_sc as plsc

_D = 32
_W = 64
_OW = 63
_NB = 16
_NR = 16
_ITEMS = _NB * _NR
_ITEMS_PER_WORKER = _ITEMS // 32
_PLANE = 4 * 4 * 8 * 128
_SCALE = np.float32(np.sqrt(32.0))


def _pos_encoding_32() -> np.ndarray:
    depth = _D / 2
    positions = np.arange(_D)[:, np.newaxis]
    depths = np.arange(depth)[np.newaxis, :] / depth
    angle_rates = 1 / 10000**depths
    angle_rads = positions * angle_rates
    return np.concatenate(
        [np.sin(angle_rads), np.cos(angle_rads)], axis=-1
    ).astype(np.float32)


_PE_BC = np.ascontiguousarray(
    np.broadcast_to(_pos_encoding_32()[:, :, None], (_D, _D, 16))
)


def _sc_body(
    x_hbm, pe_hbm, table_hbm, out_hbm, xv, gv, gv33, ov, pev, tokT, gsem, osem
):
    wid = lax.axis_index("s") * 2 + lax.axis_index("c")
    pltpu.sync_copy(pe_hbm, pev)

    iota = lax.iota(jnp.int32, 16)
    for q in range(2):
        for m in range(32):
            pv = jnp.minimum(m * 16 + iota, _OW * 8 - 1)
            i = lax.div(pv, _OW)
            tok = i * _W + (pv - i * _OW) + q
            tokT[q, pl.ds(m * 16, 16)] = tok

    def stage(t, slot):
        item = wid * _ITEMS_PER_WORKER + t
        b = item // _NR
        r = item % _NR
        pltpu.sync_copy(x_hbm.at[b, :, r], xv.at[slot])
        descs = [
            pltpu.make_async_copy(
                table_hbm.at[xv.at[slot, i]],
                gv.at[pl.ds(i * _W, _W)],
                gsem,
            )
            for i in range(8)
        ]
        for d in descs:
            d.start()
        return descs

    def repack():
        @plsc.parallel_loop(0, 8 * _W, unroll=4)
        def body(tk):
            gv33[tk, 0:16] = gv[tk, 0:16]
            gv33[tk, 16:32] = gv[tk, 16:32]

    def expand(t, slot):
        item = wid * _ITEMS_PER_WORKER + t
        r = item % _NR

        @plsc.parallel_loop(0, 2048, unroll=4)
        def body(v):
            q = v >> 10
            d = (v >> 5) & 31
            m = v & 31
            pe = pev[2 * r + q, d, 0:16]
            tokv = tokT[q, pl.ds(m * 16, 16)]
            dsplat = jnp.broadcast_to(d, (16,))
            g = plsc.load_gather(gv33, [tokv, dsplat])
            start = (
                (q << 14)
                + ((d >> 3) << 12)
                + ((d & 7) << 7)
                + ((m >> 3) << 10)
                + ((m & 7) << 4)
            )
            ov[slot, pl.ds(start, 16)] = g * _SCALE + pe

    out_descs = [None, None]
    gdescs = stage(0, 0)
    for t in range(_ITEMS_PER_WORKER):
        slot = t % 2
        for d in gdescs:
            d.wait()
        repack()
        if t + 1 < _ITEMS_PER_WORKER:
            gdescs = stage(t + 1, 1 - slot)
        if out_descs[slot] is not None:
            out_descs[slot].wait()
        expand(t, slot)
        item = wid * _ITEMS_PER_WORKER + t
        out_descs[slot] = pltpu.make_async_copy(
            ov.at[slot], out_hbm.at[pl.ds(item * 2 * _PLANE, 2 * _PLANE)], osem
        )
        out_descs[slot].start()
    for d in out_descs:
        if d is not None:
            d.wait()


@functools.partial(
    pl.kernel,
    out_type=jax.ShapeDtypeStruct((_ITEMS * 2 * _PLANE,), jnp.float32),
    mesh=plsc.VectorSubcoreMesh(core_axis_name="c", subcore_axis_name="s"),
    scratch_types=[
        pltpu.VMEM((2, 8, _W), jnp.int32),
        pltpu.VMEM((8 * _W, _D), jnp.float32),
        pltpu.VMEM((8 * _W, _D + 1), jnp.float32),
        pltpu.VMEM((2, 2 * _PLANE), jnp.float32),
        pltpu.VMEM((_D, _D, 16), jnp.float32),
        pltpu.VMEM((2, 512), jnp.int32),
        pltpu.SemaphoreType.DMA,
        pltpu.SemaphoreType.DMA,
    ],
    compiler_params=pltpu.CompilerParams(
        use_tc_tiling_on_sc=False, needs_layout_passes=False
    ),
)
def _pos_emb_sc(
    x_hbm, pe_hbm, table_hbm, out_hbm, xv, gv, gv33, ov, pev, tokT, gsem, osem
):
    _sc_body(
        x_hbm, pe_hbm, table_hbm, out_hbm, xv, gv, gv33, ov, pev, tokT, gsem, osem
    )


def kernel(x, table):
    x4 = x.reshape(_NB, 8, 16, _W)
    out1d = _pos_emb_sc(x4, _PE_BC, table)
    out6 = out1d.reshape(_NB, _D, 4, 4, 8, 128)
    t = jnp.transpose(out6, (0, 3, 5, 1, 2, 4))
    return t.reshape(_NB, 512, _D, _D)[:, : 8 * _OW]


# speedup vs baseline: 1.9322x; 1.1264x over previous
import functools

import jax
import jax.numpy as jnp
import numpy as np
from jax import lax
from jax.experimental import pallas as pl
from jax.experimental.pallas import tpu as pltpu
from jax.experimental.pallas import tpu_sc as plsc

_D = 32
_W = 64
_OW = 63
_NB = 16
_NR = 16
_ITEMS = _NB * _NR
_ITEMS_PER_WORKER = _ITEMS // 32
_PLANE = 4 * 4 * 8 * 128
_SCALE = np.float32(np.sqrt(32.0))


def _pos_encoding_32() -> np.ndarray:
    depth = _D / 2
    positions = np.arange(_D)[:, np.newaxis]
    depths = np.arange(depth)[np.newaxis, :] / depth
    angle_rates = 1 / 10000**depths
    angle_rads = positions * angle_rates
    return np.concatenate(
        [np.sin(angle_rads), np.cos(angle_rads)], axis=-1
    ).astype(np.float32)


_PE_BC = np.ascontiguousarray(
    np.broadcast_to(_pos_encoding_32()[:, :, None], (_D, _D, 16))
)


def _sc_body(
    x_hbm, pe_hbm, table_hbm, out_hbm, xv, gv, gv33, ov, pev, tokT, gsem, osem
):
    wid = lax.axis_index("s") * 2 + lax.axis_index("c")
    pltpu.sync_copy(pe_hbm, pev)

    iota = lax.iota(jnp.int32, 16)
    for q in range(2):
        for m in range(32):
            pv = jnp.minimum(m * 16 + iota, _OW * 8 - 1)
            i = lax.div(pv, _OW)
            tok = i * _W + (pv - i * _OW) + q
            tokT[q, pl.ds(m * 16, 16)] = tok

    def stage(t, slot):
        item = wid * _ITEMS_PER_WORKER + t
        b = item // _NR
        r = item % _NR
        pltpu.sync_copy(x_hbm.at[b, :, r], xv.at[slot])
        descs = [
            pltpu.make_async_copy(
                table_hbm.at[xv.at[slot, i]],
                gv.at[pl.ds(i * _W, _W)],
                gsem,
            )
            for i in range(8)
        ]
        for d in descs:
            d.start()
        return descs

    def repack():
        @plsc.parallel_loop(0, 8 * _W, unroll=4)
        def body(tk):
            gv33[tk, 0:16] = gv[tk, 0:16]
            gv33[tk, 16:32] = gv[tk, 16:32]

    def expand(t, slot):
        item = wid * _ITEMS_PER_WORKER + t
        r = item % _NR

        @plsc.parallel_loop(0, 2048, unroll=8)
        def body(v):
            q = v >> 10
            d = (v >> 5) & 31
            m = v & 31
            pe = pev[2 * r + q, d, 0:16]
            tokv = tokT[q, pl.ds(m * 16, 16)]
            dsplat = jnp.broadcast_to(d, (16,))
            g = plsc.load_gather(gv33, [tokv, dsplat])
            start = (
                (q << 14)
                + ((d >> 3) << 12)
                + ((d & 7) << 7)
                + ((m >> 3) << 10)
                + ((m & 7) << 4)
            )
            ov[slot, pl.ds(start, 16)] = g * _SCALE + pe

    out_descs = [None, None]
    gdescs = stage(0, 0)
    for t in range(_ITEMS_PER_WORKER):
        slot = t % 2
        for d in gdescs:
            d.wait()
        repack()
        if t + 1 < _ITEMS_PER_WORKER:
            gdescs = stage(t + 1, 1 - slot)
        if out_descs[slot] is not None:
            out_descs[slot].wait()
        expand(t, slot)
        item = wid * _ITEMS_PER_WORKER + t
        out_descs[slot] = pltpu.make_async_copy(
            ov.at[slot], out_hbm.at[pl.ds(item * 2 * _PLANE, 2 * _PLANE)], osem
        )
        out_descs[slot].start()
    for d in out_descs:
        if d is not None:
            d.wait()


@functools.partial(
    pl.kernel,
    out_type=jax.ShapeDtypeStruct((_ITEMS * 2 * _PLANE,), jnp.float32),
    mesh=plsc.VectorSubcoreMesh(core_axis_name="c", subcore_axis_name="s"),
    scratch_types=[
        pltpu.VMEM((2, 8, _W), jnp.int32),
        pltpu.VMEM((8 * _W, _D), jnp.float32),
        pltpu.VMEM((8 * _W, _D + 1), jnp.float32),
        pltpu.VMEM((2, 2 * _PLANE), jnp.float32),
        pltpu.VMEM((_D, _D, 16), jnp.float32),
        pltpu.VMEM((2, 512), jnp.int32),
        pltpu.SemaphoreType.DMA,
        pltpu.SemaphoreType.DMA,
    ],
    compiler_params=pltpu.CompilerParams(
        use_tc_tiling_on_sc=False, needs_layout_passes=False
    ),
)
def _pos_emb_sc(
    x_hbm, pe_hbm, table_hbm, out_hbm, xv, gv, gv33, ov, pev, tokT, gsem, osem
):
    _sc_body(
        x_hbm, pe_hbm, table_hbm, out_hbm, xv, gv, gv33, ov, pev, tokT, gsem, osem
    )


def kernel(x, table):
    x4 = x.reshape(_NB, 8, 16, _W)
    out1d = _pos_emb_sc(x4, _PE_BC, table)
    out6 = out1d.reshape(_NB, _D, 4, 4, 8, 128)
    t = jnp.transpose(out6, (0, 3, 5, 1, 2, 4))
    return t.reshape(_NB, 512, _D, _D)[:, : 8 * _OW]
